# baseline (speedup 1.0000x reference)
import functools

import jax
import jax.numpy as jnp
from jax import lax
from jax.experimental import pallas as pl
from jax.experimental.pallas import tpu as pltpu
from jax.experimental.pallas import tpu_sc as plsc

_EMB = 128
_WIN = 64
_LANES = 16
_REGS_PER_ROW = _EMB // _LANES
_CH = 128
_NP = 2


def _run(flat_idx, token_table, pos_table):
    N = flat_idx.shape[0]
    V, D = token_table.shape

    info = plsc.get_sparse_core_info()
    NC, NS = info.num_cores, info.num_subcores
    NW = NC * NS
    n_per_w = N // NW
    n_ch = n_per_w // _CH
    rounds = n_ch // _NP

    mesh = plsc.VectorSubcoreMesh(core_axis_name="c", subcore_axis_name="s")

    @functools.partial(
        pl.kernel,
        mesh=mesh,
        out_type=jax.ShapeDtypeStruct((N, D), jnp.float32),
        scratch_types=(
            [pltpu.VMEM((n_per_w,), jnp.int32),
             pltpu.VMEM((_WIN, D), jnp.float32),
             pltpu.VMEM_SHARED((V, D), jnp.float32)]
            + [pltpu.VMEM((_CH, D), jnp.float32) for _ in range(2 * _NP)]
            + [pltpu.SemaphoreType.DMA for _ in range(2 * _NP)]
        ),
    )
    def k(idx_hbm, tok_hbm, pos_hbm, out_hbm, idx_all, pos_v, tab_sh,
          *bufs_and_sems):
        gbuf = list(bufs_and_sems[:_NP])
        obuf = list(bufs_and_sems[_NP:2 * _NP])
        gsem = list(bufs_and_sems[2 * _NP:3 * _NP])
        osem = list(bufs_and_sems[3 * _NP:])

        sid = lax.axis_index("s")
        wid = sid * NC + lax.axis_index("c")
        base = wid * n_per_w
        v_per_s = V // NS
        pltpu.sync_copy(tok_hbm.at[pl.ds(sid * v_per_s, v_per_s)],
                        tab_sh.at[pl.ds(sid * v_per_s, v_per_s)])
        pltpu.sync_copy(pos_hbm, pos_v)
        pltpu.sync_copy(idx_hbm.at[pl.ds(base, n_per_w)], idx_all)
        plsc.subcore_barrier()

        def gather_copy(lci, p):
            src = tab_sh.at[idx_all.at[pl.ds(lci * _CH, _CH)]]
            return pltpu.make_async_copy(src, gbuf[p], gsem[p])

        def out_copy(lci, p):
            return pltpu.make_async_copy(
                obuf[p], out_hbm.at[pl.ds(base + lci * _CH, _CH)], osem[p])

        for p in range(_NP):
            gather_copy(p, p).start()

        def round_body(i, _):
            for p in range(_NP):
                lci = i * _NP + p
                gather_copy(lci, p).wait()

                @pl.when(i >= 1)
                def _():
                    out_copy(lci - _NP, p).wait()

                gb, ob = gbuf[p], obuf[p]

                def w_body(wi, _):
                    for u in range(2):
                        w = wi * 2 + u
                        for kk in range(_REGS_PER_ROW):
                            sl = pl.ds(kk * _LANES, _LANES)
                            pv = pos_v[w, sl]
                            for r in range(_CH // _WIN):
                                row = r * _WIN + w
                                ob[row, sl] = gb[row, sl] + pv
                    return 0

                lax.fori_loop(0, _WIN // 2, w_body, 0)
                out_copy(lci, p).start()

                @pl.when(i < rounds - 1)
                def _():
                    gather_copy(lci + _NP, p).start()

            return 0

        lax.fori_loop(0, rounds, round_body, 0)
        for p in range(_NP):
            out_copy((rounds - 1) * _NP + p, p).wait()

    return k(flat_idx, token_table, pos_table)


def kernel(input, token_table, pos_table):
    B, W = input.shape
    D = token_table.shape[1]
    flat_idx = input.reshape(B * W).astype(jnp.int32)
    out = _run(flat_idx, token_table, pos_table)
    return out.reshape(B, W, D)


# speedup vs baseline: 1.0070x; 1.0070x over previous
import functools

import jax
import jax.numpy as jnp
from jax import lax
from jax.experimental import pallas as pl
from jax.experimental.pallas import tpu as pltpu
from jax.experimental.pallas import tpu_sc as plsc

_EMB = 128
_WIN = 64
_LANES = 16
_REGS_PER_ROW = _EMB // _LANES
_CH = 128
_NB = 4
_LEAD = 2


def _run(flat_idx, token_table, pos_table):
    N = flat_idx.shape[0]
    V, D = token_table.shape

    info = plsc.get_sparse_core_info()
    NC, NS = info.num_cores, info.num_subcores
    NW = NC * NS
    n_per_w = N // NW
    n_ch = n_per_w // _CH
    rounds = n_ch // _NB

    mesh = plsc.VectorSubcoreMesh(core_axis_name="c", subcore_axis_name="s")

    @functools.partial(
        pl.kernel,
        mesh=mesh,
        out_type=jax.ShapeDtypeStruct((N, D), jnp.float32),
        scratch_types=(
            [pltpu.VMEM((n_per_w,), jnp.int32),
             pltpu.VMEM((_WIN, D), jnp.float32),
             pltpu.VMEM_SHARED((V, D), jnp.float32)]
            + [pltpu.VMEM((_CH, D), jnp.float32) for _ in range(_NB)]
            + [pltpu.SemaphoreType.DMA for _ in range(2 * _NB)]
        ),
    )
    def k(idx_hbm, tok_hbm, pos_hbm, out_hbm, idx_all, pos_v, tab_sh,
          *bufs_and_sems):
        rows = list(bufs_and_sems[:_NB])
        gsem = list(bufs_and_sems[_NB:2 * _NB])
        osem = list(bufs_and_sems[2 * _NB:])

        sid = lax.axis_index("s")
        wid = sid * NC + lax.axis_index("c")
        base = wid * n_per_w
        v_per_s = V // NS
        pltpu.sync_copy(tok_hbm.at[pl.ds(sid * v_per_s, v_per_s)],
                        tab_sh.at[pl.ds(sid * v_per_s, v_per_s)])
        pltpu.sync_copy(pos_hbm, pos_v)
        pltpu.sync_copy(idx_hbm.at[pl.ds(base, n_per_w)], idx_all)
        plsc.subcore_barrier()

        def gather_copy(lci, b):
            src = tab_sh.at[idx_all.at[pl.ds(lci * _CH, _CH)]]
            return pltpu.make_async_copy(src, rows[b], gsem[b])

        def out_copy(lci, b):
            return pltpu.make_async_copy(
                rows[b], out_hbm.at[pl.ds(base + lci * _CH, _CH)], osem[b])

        for b in range(_NB):
            gather_copy(b, b).start()

        def round_body(i, _):
            for b in range(_NB):
                lci = i * _NB + b
                gather_copy(lci, b).wait()
                rows_b = rows[b]

                def w_body(wi, _):
                    for u in range(2):
                        w = wi * 2 + u
                        for kk in range(_REGS_PER_ROW):
                            sl = pl.ds(kk * _LANES, _LANES)
                            pv = pos_v[w, sl]
                            for r in range(_CH // _WIN):
                                row = r * _WIN + w
                                plsc.addupdate(rows_b.at[row, sl], pv)
                    return 0

                lax.fori_loop(0, _WIN // 2, w_body, 0)
                out_copy(lci, b).start()

                bn = (b + _LEAD) % _NB
                cond = (i >= 1) if b < _LEAD else (i < rounds - 1)

                @pl.when(cond)
                def _():
                    out_copy(lci + _LEAD - _NB, bn).wait()
                    gather_copy(lci + _LEAD, bn).start()

            return 0

        lax.fori_loop(0, rounds, round_body, 0)
        for b in range(_NB):
            out_copy((rounds - 1) * _NB + b, b).wait()

    return k(flat_idx, token_table, pos_table)


def kernel(input, token_table, pos_table):
    B, W = input.shape
    D = token_table.shape[1]
    flat_idx = input.reshape(B * W).astype(jnp.int32)
    out = _run(flat_idx, token_table, pos_table)
    return out.reshape(B, W, D)
